# baseline (speedup 1.0000x reference)
import functools

import jax
import jax.numpy as jnp
from jax import lax
from jax.experimental import pallas as pl
from jax.experimental.pallas import tpu as pltpu
from jax.experimental.pallas import tpu_sc as plsc

E = 160000
C = 128
E2 = E // 2
NC, NS = 2, 16
NW = NC * NS
CH = 64
CH2 = CH // 2
NCHUNKS = E // CH
CHUNKS_PER_W = -(-NCHUNKS // NW)
NPAIRS = -(-CHUNKS_PER_W // 2)
T2 = 320


def _pack_bf16_pair(va, vb):
    ua = plsc.bitcast(va, jnp.int32) + 0x8000
    ub = plsc.bitcast(vb, jnp.int32) + 0x8000
    lo = lax.shift_right_logical(ua, 16)
    hi = ub & jnp.int32(-65536)
    return lo | hi


def _sc_gather_combine(xt, idx):
    mesh = plsc.VectorSubcoreMesh(
        core_axis_name="c", subcore_axis_name="s",
        num_cores=NC, num_subcores=NS)

    @functools.partial(
        pl.kernel,
        out_type=jax.ShapeDtypeStruct((4, E2, C), jnp.int32),
        mesh=mesh,
        scratch_types=(
            [pltpu.VMEM((4 * CH,), jnp.int32) for _ in range(2)]
            + [pltpu.VMEM((CH, C), jnp.float32) for _ in range(8)]
            + [pltpu.VMEM((CH2, C), jnp.int32) for _ in range(8)]
            + [pltpu.SemaphoreType.DMA for _ in range(4)]
        ),
        compiler_params=pltpu.CompilerParams(needs_layout_passes=False),
    )
    def gather_kernel(xt_hbm, idx_hbm, g_hbm, ix0, ix1,
                      f00, f01, f02, f03, f10, f11, f12, f13,
                      o00, o01, o02, o03, o10, o11, o12, o13,
                      gs0, gs1, ws0, ws1):
        idxb = (ix0, ix1)
        fin = ((f00, f01, f02, f03), (f10, f11, f12, f13))
        out = ((o00, o01, o02, o03), (o10, o11, o12, o13))
        gsem = (gs0, gs1)
        wsem = (ws0, ws1)
        wid = lax.axis_index("s") * NC + lax.axis_index("c")

        def pair_body(p, carry):
            for s in range(2):
                chunk = wid + (2 * p + s) * NW

                @pl.when(chunk < NCHUNKS)
                def _(s=s, chunk=chunk):
                    @pl.when(p > 0)
                    def _():
                        for k in range(4):
                            pltpu.make_async_copy(
                                out[s][k], g_hbm.at[k, pl.ds(0, CH2)],
                                wsem[s]).wait()
                    pltpu.sync_copy(idx_hbm.at[chunk], idxb[s])
                    for k in range(4):
                        pltpu.async_copy(
                            xt_hbm.at[idxb[s].at[pl.ds(CH * k, CH)]],
                            fin[s][k], gsem[s])

            for s in range(2):
                chunk = wid + (2 * p + s) * NW

                @pl.when(chunk < NCHUNKS)
                def _(s=s, chunk=chunk):
                    for k in range(4):
                        pltpu.make_async_copy(
                            xt_hbm.at[idxb[s].at[pl.ds(CH * k, CH)]],
                            fin[s][k], gsem[s]).wait()
                    y1, y2, y3, y4 = fin[s]
                    os1, os2, od1, od2 = out[s]

                    def conv_body(rr, c2):
                        for h in range(2):
                            r = 2 * rr + h
                            for q in range(4):
                                sa = pl.ds(32 * q, 16)
                                sb = pl.ds(32 * q + 16, 16)
                                dst = pl.ds(64 * h + 16 * q, 16)
                                a1, b1 = y1[r, sa], y1[r, sb]
                                a2, b2 = y2[r, sa], y2[r, sb]
                                a3, b3 = y3[r, sa], y3[r, sb]
                                a4, b4 = y4[r, sa], y4[r, sb]
                                os1[rr, dst] = _pack_bf16_pair(
                                    a1 + a3, b1 + b3)
                                os2[rr, dst] = _pack_bf16_pair(
                                    a2 + a4, b2 + b4)
                                od1[rr, dst] = _pack_bf16_pair(
                                    jnp.abs(a1 - a3), jnp.abs(b1 - b3))
                                od2[rr, dst] = _pack_bf16_pair(
                                    jnp.abs(a2 - a4), jnp.abs(b2 - b4))
                        return c2

                    lax.fori_loop(0, CH2, conv_body, 0)
                    gbase = chunk * CH2
                    for k in range(4):
                        pltpu.async_copy(out[s][k],
                                         g_hbm.at[k, pl.ds(gbase, CH2)],
                                         wsem[s])

            return carry

        lax.fori_loop(0, NPAIRS, pair_body, 0)

        for s in range(2):
            chunk = wid + (2 * (NPAIRS - 1) + s) * NW

            @pl.when(chunk < NCHUNKS)
            def _(s=s):
                for k in range(4):
                    pltpu.make_async_copy(
                        out[s][k], g_hbm.at[k, pl.ds(0, CH2)],
                        wsem[s]).wait()

    return gather_kernel(xt, idx)


def _tc_body(xt2_ref, g_ref, wcat_ref, b2_ref, out_ref):
    planes = []
    for k in range(4):
        gk = g_ref[k]
        lo = pltpu.unpack_elementwise(
            gk, index=0, packed_dtype=jnp.bfloat16,
            unpacked_dtype=jnp.float32).astype(jnp.bfloat16)
        hi = pltpu.unpack_elementwise(
            gk, index=1, packed_dtype=jnp.bfloat16,
            unpacked_dtype=jnp.float32).astype(jnp.bfloat16)
        planes.append((lo, hi))
    for h in range(2):
        ops = [xt2_ref[:, pl.ds(C * h, C)].astype(jnp.bfloat16)]
        for lo, hi in planes:
            ops.append(lo[:, C // 2 * h:C // 2 * (h + 1)])
            ops.append(hi[:, C // 2 * h:C // 2 * (h + 1)])
        cat = jnp.concatenate(ops, axis=1)
        acc = jnp.dot(cat, wcat_ref[...],
                      preferred_element_type=jnp.float32)
        out_ref[:, pl.ds(C * h, C)] = acc + b2_ref[:, pl.ds(C * h, C)]


def _tc_conv(xt2, g, wcat, b2):
    return pl.pallas_call(
        _tc_body,
        grid=(E2 // T2,),
        in_specs=[
            pl.BlockSpec((T2, 2 * C), lambda i: (i, 0)),
            pl.BlockSpec((4, T2, C), lambda i: (0, i, 0)),
            pl.BlockSpec((5 * C, C), lambda i: (0, 0)),
            pl.BlockSpec((1, 2 * C), lambda i: (0, 0)),
        ],
        out_specs=pl.BlockSpec((T2, 2 * C), lambda i: (i, 0)),
        out_shape=jax.ShapeDtypeStruct((E2, 2 * C), jnp.float32),
    )(xt2, g, wcat, b2)


def kernel(x, gemm_edges, W, b):
    xt = x[0, :, :, 0].T
    idx = (gemm_edges[0].astype(jnp.int32)
           .reshape(NCHUNKS, CH, 4).transpose(0, 2, 1).reshape(NCHUNKS, 4 * CH))
    p64 = jnp.arange(C // 2)
    perm_lo = 32 * (p64 // 16) + (p64 % 16)
    perm_hi = perm_lo + 16
    wt = jnp.transpose(W[:, :, 0, :], (2, 1, 0)).astype(jnp.bfloat16)
    rows = [wt[0]]
    for k in range(1, 5):
        rows.append(wt[k][perm_lo, :])
        rows.append(wt[k][perm_hi, :])
    wcat = jnp.concatenate(rows, axis=0)
    g = _sc_gather_combine(xt, idx)
    xt2 = xt.reshape(E2, 2 * C)
    b2 = jnp.tile(b, 2)[None, :]
    out2 = _tc_conv(xt2, g, wcat, b2)
    return out2.reshape(E, C).T[None, :, :, None]


# speedup vs baseline: 1.1019x; 1.1019x over previous
import functools

import jax
import jax.numpy as jnp
from jax import lax
from jax.experimental import pallas as pl
from jax.experimental.pallas import tpu as pltpu
from jax.experimental.pallas import tpu_sc as plsc

E = 160000
C = 128
E2 = E // 2
NC, NS = 2, 16
NW = NC * NS
CH = 64
CH2 = CH // 2
NCHUNKS = E // CH
CHUNKS_PER_W = -(-NCHUNKS // NW)
NPAIRS = -(-CHUNKS_PER_W // 2)
T2 = 320


def _pack_bf16_pair(va, vb):
    ua = plsc.bitcast(va, jnp.int32) + 0x8000
    ub = plsc.bitcast(vb, jnp.int32) + 0x8000
    lo = lax.shift_right_logical(ua, 16)
    hi = ub & jnp.int32(-65536)
    return lo | hi


def _sc_gather_combine(xt, idx):
    mesh = plsc.VectorSubcoreMesh(
        core_axis_name="c", subcore_axis_name="s",
        num_cores=NC, num_subcores=NS)

    @functools.partial(
        pl.kernel,
        out_type=jax.ShapeDtypeStruct((4, E2, C), jnp.int32),
        mesh=mesh,
        scratch_types=(
            [pltpu.VMEM((4 * CH,), jnp.int32) for _ in range(4)]
            + [pltpu.VMEM((CH, C), jnp.float32) for _ in range(8)]
            + [pltpu.VMEM((CH2, C), jnp.int32) for _ in range(8)]
            + [pltpu.SemaphoreType.DMA for _ in range(8)]
        ),
        compiler_params=pltpu.CompilerParams(needs_layout_passes=False),
    )
    def gather_kernel(xt_hbm, idx_hbm, g_hbm, ix0, ix1, ix2, ix3,
                      f00, f01, f02, f03, f10, f11, f12, f13,
                      o00, o01, o02, o03, o10, o11, o12, o13,
                      gs0, gs1, ws0, ws1, is0, is1, is2, is3):
        idxb = (ix0, ix1, ix2, ix3)
        fin = ((f00, f01, f02, f03), (f10, f11, f12, f13))
        out = ((o00, o01, o02, o03), (o10, o11, o12, o13))
        gsem = (gs0, gs1)
        wsem = (ws0, ws1)
        isem = (is0, is1, is2, is3)
        wid = lax.axis_index("s") * NC + lax.axis_index("c")

        def chunk_of(p, s):
            return wid + (2 * p + s) * NW

        for s in range(2):
            @pl.when(chunk_of(0, s) < NCHUNKS)
            def _(s=s):
                pltpu.async_copy(idx_hbm.at[chunk_of(0, s)], idxb[s],
                                 isem[s])

        def quad_body(qq, carry):
            for pp in range(2):
                p = 2 * qq + pp
                for s in range(2):
                    a = 2 * pp + s

                    @pl.when(chunk_of(p, s) < NCHUNKS)
                    def _(s=s, a=a, p=p):
                        pltpu.make_async_copy(
                            idx_hbm.at[0], idxb[a], isem[a]).wait()
                        for k in range(4):
                            pltpu.async_copy(
                                xt_hbm.at[idxb[a].at[pl.ds(CH * k, CH)]],
                                fin[s][k], gsem[s])

                for s in range(2):
                    a2 = 2 * ((pp + 1) % 2) + s

                    @pl.when(chunk_of(p + 1, s) < NCHUNKS)
                    def _(s=s, a2=a2, p=p):
                        pltpu.async_copy(idx_hbm.at[chunk_of(p + 1, s)],
                                         idxb[a2], isem[a2])

                for s in range(2):
                    a = 2 * pp + s

                    @pl.when(chunk_of(p, s) < NCHUNKS)
                    def _(s=s, a=a, p=p):
                        for k in range(4):
                            pltpu.make_async_copy(
                                xt_hbm.at[idxb[a].at[pl.ds(CH * k, CH)]],
                                fin[s][k], gsem[s]).wait()
                        @pl.when(p > 0)
                        def _():
                            for k in range(4):
                                pltpu.make_async_copy(
                                    out[s][k], g_hbm.at[k, pl.ds(0, CH2)],
                                    wsem[s]).wait()
                        y1, y2, y3, y4 = fin[s]
                        os1, os2, od1, od2 = out[s]

                        def conv_body(rr, c2):
                            for h in range(2):
                                r = 2 * rr + h
                                for q in range(4):
                                    sa = pl.ds(32 * q, 16)
                                    sb = pl.ds(32 * q + 16, 16)
                                    dst = pl.ds(64 * h + 16 * q, 16)
                                    a1, b1 = y1[r, sa], y1[r, sb]
                                    a2, b2 = y2[r, sa], y2[r, sb]
                                    a3, b3 = y3[r, sa], y3[r, sb]
                                    a4, b4 = y4[r, sa], y4[r, sb]
                                    os1[rr, dst] = _pack_bf16_pair(
                                        a1 + a3, b1 + b3)
                                    os2[rr, dst] = _pack_bf16_pair(
                                        a2 + a4, b2 + b4)
                                    od1[rr, dst] = _pack_bf16_pair(
                                        jnp.abs(a1 - a3), jnp.abs(b1 - b3))
                                    od2[rr, dst] = _pack_bf16_pair(
                                        jnp.abs(a2 - a4), jnp.abs(b2 - b4))
                            return c2

                        lax.fori_loop(0, CH2, conv_body, 0)
                        gbase = chunk_of(p, s) * CH2
                        for k in range(4):
                            pltpu.async_copy(out[s][k],
                                             g_hbm.at[k, pl.ds(gbase, CH2)],
                                             wsem[s])

            return carry

        lax.fori_loop(0, NPAIRS // 2, quad_body, 0)

        for s in range(2):
            @pl.when(chunk_of(0, s) < NCHUNKS)
            def _(s=s):
                for k in range(4):
                    pltpu.make_async_copy(
                        out[s][k], g_hbm.at[k, pl.ds(0, CH2)],
                        wsem[s]).wait()

    return gather_kernel(xt, idx)


def _tc_body(xt2_ref, g_ref, wcat_ref, b_ref, out_ref):
    planes = []
    for k in range(4):
        gk = g_ref[k]
        lo = lax.bitcast_convert_type(
            lax.shift_left(gk, 16), jnp.float32).astype(jnp.bfloat16)
        hi = lax.bitcast_convert_type(
            gk & jnp.int32(-65536), jnp.float32).astype(jnp.bfloat16)
        planes.append((lo, hi))
    for h in range(2):
        ops = [xt2_ref[:, pl.ds(C * h, C)].astype(jnp.bfloat16)]
        for lo, hi in planes:
            ops.append(lo[:, C // 2 * h:C // 2 * (h + 1)])
            ops.append(hi[:, C // 2 * h:C // 2 * (h + 1)])
        cat = jnp.concatenate(ops, axis=1)
        acc = jnp.dot(cat, wcat_ref[...],
                      preferred_element_type=jnp.float32)
        out_ref[h] = acc + b_ref[...]


def _tc_conv(xt2, g, wcat, b_row):
    return pl.pallas_call(
        _tc_body,
        grid=(E2 // T2,),
        in_specs=[
            pl.BlockSpec((T2, 2 * C), lambda i: (i, 0)),
            pl.BlockSpec((4, T2, C), lambda i: (0, i, 0)),
            pl.BlockSpec((5 * C, C), lambda i: (0, 0)),
            pl.BlockSpec((1, C), lambda i: (0, 0)),
        ],
        out_specs=pl.BlockSpec((2, T2, C), lambda i: (0, i, 0)),
        out_shape=jax.ShapeDtypeStruct((2, E2, C), jnp.float32),
    )(xt2, g, wcat, b_row)


def kernel(x, gemm_edges, W, b):
    xt = x[0, :, :, 0].T
    idx = (gemm_edges[0].astype(jnp.int32)
           .reshape(NCHUNKS, CH, 4).transpose(0, 2, 1).reshape(NCHUNKS, 4 * CH))
    p64 = jnp.arange(C // 2)
    perm_lo = 32 * (p64 // 16) + (p64 % 16)
    perm_hi = perm_lo + 16
    wt = jnp.transpose(W[:, :, 0, :], (2, 1, 0)).astype(jnp.bfloat16)
    rows = [wt[0]]
    for k in range(1, 5):
        rows.append(wt[k][perm_lo, :])
        rows.append(wt[k][perm_hi, :])
    wcat = jnp.concatenate(rows, axis=0)
    g = _sc_gather_combine(xt, idx)
    xt2 = xt.reshape(E2, 2 * C)
    out3 = _tc_conv(xt2, g, wcat, b[None, :])
    return jnp.transpose(out3, (2, 1, 0)).reshape(C, E)[None, :, :, None]


# speedup vs baseline: 1.4917x; 1.3537x over previous
import functools

import jax
import jax.numpy as jnp
from jax import lax
from jax.experimental import pallas as pl
from jax.experimental.pallas import tpu as pltpu
from jax.experimental.pallas import tpu_sc as plsc

E = 160000
C = 128
NC, NS = 2, 16
NW = NC * NS
CH = 32
NCHUNKS = E // CH
NOCT = -(-(-(-NCHUNKS // NW)) // 8)
TE = 640


def _sc_gather(xt, idx):
    mesh = plsc.VectorSubcoreMesh(
        core_axis_name="c", subcore_axis_name="s",
        num_cores=NC, num_subcores=NS)

    @functools.partial(
        pl.kernel,
        out_type=jax.ShapeDtypeStruct((4, E, C), jnp.float32),
        mesh=mesh,
        scratch_types=(
            [pltpu.VMEM((4 * CH,), jnp.int32) for _ in range(8)]
            + [pltpu.VMEM((CH, C), jnp.float32) for _ in range(16)]
            + [pltpu.SemaphoreType.DMA for _ in range(16)]
        ),
        compiler_params=pltpu.CompilerParams(needs_layout_passes=False),
    )
    def gather_kernel(xt_hbm, idx_hbm, g_hbm, *scr):
        idxb = scr[0:8]
        fin = tuple(tuple(scr[8 + 4 * s + k] for k in range(4))
                    for s in range(4))
        gsem = scr[24:28]
        wsem = scr[28:32]
        isem = scr[32:40]
        wid = lax.axis_index("s") * NC + lax.axis_index("c")

        def chunk_of(j):
            return wid + j * NW

        for s in range(4):
            @pl.when(chunk_of(s) < NCHUNKS)
            def _(s=s):
                pltpu.async_copy(idx_hbm.at[chunk_of(s)], idxb[s], isem[s])

        def oct_body(oo, carry):
            for half in range(2):
                for s in range(4):
                    j = 8 * oo + 4 * half + s
                    a = 4 * half + s
                    chunk = chunk_of(j)

                    @pl.when(chunk < NCHUNKS)
                    def _(s=s, a=a, j=j, chunk=chunk):
                        pltpu.make_async_copy(
                            idx_hbm.at[0], idxb[a], isem[a]).wait()
                        @pl.when(j >= 4)
                        def _():
                            for k in range(4):
                                pltpu.make_async_copy(
                                    fin[s][k], g_hbm.at[k, pl.ds(0, CH)],
                                    wsem[s]).wait()
                        for k in range(4):
                            pltpu.async_copy(
                                xt_hbm.at[idxb[a].at[pl.ds(CH * k, CH)]],
                                fin[s][k], gsem[s])

                for s in range(4):
                    jn = 8 * oo + 4 * (half + 1) + s
                    a2 = 4 * ((half + 1) % 2) + s

                    @pl.when(chunk_of(jn) < NCHUNKS)
                    def _(s=s, a2=a2, jn=jn):
                        pltpu.async_copy(idx_hbm.at[chunk_of(jn)],
                                         idxb[a2], isem[a2])

                for s in range(4):
                    j = 8 * oo + 4 * half + s
                    a = 4 * half + s
                    chunk = chunk_of(j)

                    @pl.when(chunk < NCHUNKS)
                    def _(s=s, a=a, chunk=chunk):
                        for k in range(4):
                            pltpu.make_async_copy(
                                xt_hbm.at[idxb[a].at[pl.ds(CH * k, CH)]],
                                fin[s][k], gsem[s]).wait()
                        base = chunk * CH
                        for k in range(4):
                            pltpu.async_copy(fin[s][k],
                                             g_hbm.at[k, pl.ds(base, CH)],
                                             wsem[s])

            return carry

        lax.fori_loop(0, NOCT, oct_body, 0)

        for s in range(4):
            @pl.when(chunk_of(s) < NCHUNKS)
            def _(s=s):
                for k in range(4):
                    pltpu.make_async_copy(
                        fin[s][k], g_hbm.at[k, pl.ds(0, CH)],
                        wsem[s]).wait()

    return gather_kernel(xt, idx)


def _tc_body(xt_ref, g_ref, wt_ref, b_ref, out_ref):
    y1 = g_ref[0]
    y2 = g_ref[1]
    y3 = g_ref[2]
    y4 = g_ref[3]
    s1 = y1 + y3
    s2 = y2 + y4
    d1 = jnp.abs(y1 - y3)
    d2 = jnp.abs(y2 - y4)
    acc = jnp.dot(xt_ref[...], wt_ref[0], preferred_element_type=jnp.float32)
    acc = acc + jnp.dot(s1, wt_ref[1], preferred_element_type=jnp.float32)
    acc = acc + jnp.dot(s2, wt_ref[2], preferred_element_type=jnp.float32)
    acc = acc + jnp.dot(d1, wt_ref[3], preferred_element_type=jnp.float32)
    acc = acc + jnp.dot(d2, wt_ref[4], preferred_element_type=jnp.float32)
    out_ref[...] = acc + b_ref[...]


def _tc_conv(xt, g, wt, b_row):
    return pl.pallas_call(
        _tc_body,
        grid=(E // TE,),
        in_specs=[
            pl.BlockSpec((TE, C), lambda i: (i, 0)),
            pl.BlockSpec((4, TE, C), lambda i: (0, i, 0)),
            pl.BlockSpec((5, C, C), lambda i: (0, 0, 0)),
            pl.BlockSpec((1, C), lambda i: (0, 0)),
        ],
        out_specs=pl.BlockSpec((TE, C), lambda i: (i, 0)),
        out_shape=jax.ShapeDtypeStruct((E, C), jnp.float32),
    )(xt, g, wt, b_row)


def kernel(x, gemm_edges, W, b):
    xt = x[0, :, :, 0].T
    idx = (gemm_edges[0].astype(jnp.int32)
           .reshape(NCHUNKS, CH, 4).transpose(0, 2, 1)
           .reshape(NCHUNKS, 4 * CH))
    wt = jnp.transpose(W[:, :, 0, :], (2, 1, 0))
    g = _sc_gather(xt, idx)
    out_t = _tc_conv(xt, g, wt, b[None, :])
    return out_t.T[None, :, :, None]


# speedup vs baseline: 1.6817x; 1.1274x over previous
import functools

import jax
import jax.numpy as jnp
from jax import lax
from jax.experimental import pallas as pl
from jax.experimental.pallas import tpu as pltpu
from jax.experimental.pallas import tpu_sc as plsc

E = 160000
C = 128
NC, NS = 2, 16
NW = NC * NS
CH = 32
NCHUNKS = E // CH
NOCT = -(-(-(-NCHUNKS // NW)) // 8)
TE = 1280


def _sc_gather(xt, idx):
    mesh = plsc.VectorSubcoreMesh(
        core_axis_name="c", subcore_axis_name="s",
        num_cores=NC, num_subcores=NS)

    @functools.partial(
        pl.kernel,
        out_type=jax.ShapeDtypeStruct((4, E, C), jnp.float32),
        mesh=mesh,
        scratch_types=(
            [pltpu.VMEM((4 * CH,), jnp.int32) for _ in range(8)]
            + [pltpu.VMEM((CH, C), jnp.float32) for _ in range(16)]
            + [pltpu.SemaphoreType.DMA for _ in range(16)]
        ),
        compiler_params=pltpu.CompilerParams(needs_layout_passes=False),
    )
    def gather_kernel(xt_hbm, idx_hbm, g_hbm, *scr):
        idxb = scr[0:8]
        fin = tuple(tuple(scr[8 + 4 * s + k] for k in range(4))
                    for s in range(4))
        gsem = scr[24:28]
        wsem = scr[28:32]
        isem = scr[32:40]
        wid = lax.axis_index("s") * NC + lax.axis_index("c")

        def chunk_of(j):
            return wid + j * NW

        for s in range(4):
            @pl.when(chunk_of(s) < NCHUNKS)
            def _(s=s):
                pltpu.async_copy(idx_hbm.at[chunk_of(s)], idxb[s], isem[s])

        def oct_body(oo, carry):
            for half in range(2):
                for s in range(4):
                    j = 8 * oo + 4 * half + s
                    a = 4 * half + s
                    chunk = chunk_of(j)

                    @pl.when(chunk < NCHUNKS)
                    def _(s=s, a=a, j=j, chunk=chunk):
                        pltpu.make_async_copy(
                            idx_hbm.at[0], idxb[a], isem[a]).wait()
                        @pl.when(j >= 4)
                        def _():
                            for k in range(4):
                                pltpu.make_async_copy(
                                    fin[s][k], g_hbm.at[k, pl.ds(0, CH)],
                                    wsem[s]).wait()
                        for k in range(4):
                            pltpu.async_copy(
                                xt_hbm.at[idxb[a].at[pl.ds(CH * k, CH)]],
                                fin[s][k], gsem[s])

                for s in range(4):
                    jn = 8 * oo + 4 * (half + 1) + s
                    a2 = 4 * ((half + 1) % 2) + s

                    @pl.when(chunk_of(jn) < NCHUNKS)
                    def _(s=s, a2=a2, jn=jn):
                        pltpu.async_copy(idx_hbm.at[chunk_of(jn)],
                                         idxb[a2], isem[a2])

                for s in range(4):
                    j = 8 * oo + 4 * half + s
                    a = 4 * half + s
                    chunk = chunk_of(j)

                    @pl.when(chunk < NCHUNKS)
                    def _(s=s, a=a, chunk=chunk):
                        for k in range(4):
                            pltpu.make_async_copy(
                                xt_hbm.at[idxb[a].at[pl.ds(CH * k, CH)]],
                                fin[s][k], gsem[s]).wait()
                        base = chunk * CH
                        for k in range(4):
                            pltpu.async_copy(fin[s][k],
                                             g_hbm.at[k, pl.ds(base, CH)],
                                             wsem[s])

            return carry

        lax.fori_loop(0, NOCT, oct_body, 0)

        for s in range(4):
            @pl.when(chunk_of(s) < NCHUNKS)
            def _(s=s):
                for k in range(4):
                    pltpu.make_async_copy(
                        fin[s][k], g_hbm.at[k, pl.ds(0, CH)],
                        wsem[s]).wait()

    return gather_kernel(xt, idx)


def _tc_body(xt_ref, g_ref, wt_ref, b_ref, out_ref):
    y1 = g_ref[0]
    y2 = g_ref[1]
    y3 = g_ref[2]
    y4 = g_ref[3]
    s1 = y1 + y3
    s2 = y2 + y4
    d1 = jnp.abs(y1 - y3)
    d2 = jnp.abs(y2 - y4)
    dn = (((1,), (1,)), ((), ()))
    acc = lax.dot_general(wt_ref[0], xt_ref[...], dn,
                          preferred_element_type=jnp.float32)
    acc = acc + lax.dot_general(wt_ref[1], s1, dn,
                                preferred_element_type=jnp.float32)
    acc = acc + lax.dot_general(wt_ref[2], s2, dn,
                                preferred_element_type=jnp.float32)
    acc = acc + lax.dot_general(wt_ref[3], d1, dn,
                                preferred_element_type=jnp.float32)
    acc = acc + lax.dot_general(wt_ref[4], d2, dn,
                                preferred_element_type=jnp.float32)
    out_ref[...] = acc + b_ref[...]


def _tc_conv(xt, g, wt, b_col):
    return pl.pallas_call(
        _tc_body,
        grid=(E // TE,),
        in_specs=[
            pl.BlockSpec((TE, C), lambda i: (i, 0)),
            pl.BlockSpec((4, TE, C), lambda i: (0, i, 0)),
            pl.BlockSpec((5, C, C), lambda i: (0, 0, 0)),
            pl.BlockSpec((C, 1), lambda i: (0, 0)),
        ],
        out_specs=pl.BlockSpec((C, TE), lambda i: (0, i)),
        out_shape=jax.ShapeDtypeStruct((C, E), jnp.float32),
    )(xt, g, wt, b_col)


def kernel(x, gemm_edges, W, b):
    xt = x[0, :, :, 0].T
    idx = (gemm_edges[0].astype(jnp.int32)
           .reshape(NCHUNKS, CH, 4).transpose(0, 2, 1)
           .reshape(NCHUNKS, 4 * CH))
    wt = W[:, :, 0, :].transpose(2, 0, 1)
    g = _sc_gather(xt, idx)
    out = _tc_conv(xt, g, wt, b[:, None])
    return out[None, :, :, None]
